# baseline (speedup 1.0000x reference)
import functools

import jax
import jax.numpy as jnp
from jax import lax
from jax.experimental import pallas as pl
from jax.experimental.pallas import tpu as pltpu
from jax.experimental.pallas import tpu_sc as plsc

_N_ATOMS = 32768
_N_MOL = 16
_NODE_DIM = 128
_HIDDEN_DIM = 64
_N_ELEM = 119
_TILE = 8192
_GRID = _N_ATOMS // _TILE

_NC = 2
_NS = 16
_NW = _NC * _NS
_CHUNK = _N_ATOMS // _NW
_LANES = 16


def _mass_side_kernel(feat_hbm, atno_hbm, mt_hbm, out_hbm,
                      feat_v, atno_v, m_v, mt_v, acc_v, sem):
    cid = lax.axis_index("c")
    sid = lax.axis_index("s")
    wid = cid * _NS + sid
    base = wid * _CHUNK

    copies = [
        pltpu.async_copy(atno_hbm.at[pl.ds(base, _CHUNK)], atno_v, sem),
        pltpu.async_copy(feat_hbm.at[:, pl.ds(base, _CHUNK)], feat_v, sem),
        pltpu.async_copy(mt_hbm, mt_v.at[pl.ds(0, _N_ELEM)], sem),
    ]
    for c in copies:
        c.wait()

    tab = [mt_v[pl.ds(t * _LANES, _LANES)] for t in range(8)]

    def gather_body(j, _):
        sl = pl.ds(j * _LANES, _LANES)
        a = atno_v[sl]
        hi = lax.shift_right_logical(a, 4)
        lo = lax.bitwise_and(a, 15)
        m = jnp.zeros((_LANES,), jnp.float32)
        for t in range(8):
            m = jnp.where(hi == t,
                          tab[t].at[lo].get(mode="promise_in_bounds"), m)
        m_v[sl] = m
        return 0

    lax.fori_loop(0, _CHUNK // _LANES, gather_body, 0)

    for q in range(4):
        for mol in range(_N_MOL):
            acc_v[q, mol, :] = jnp.zeros((_LANES,), jnp.float32)

    bf0 = feat_v[4, pl.ds(0, _LANES)][0]
    bf1 = feat_v[4, pl.ds(_CHUNK - _LANES, _LANES)][_LANES - 1]
    bmin = bf0.astype(jnp.int32)
    bmax = bf1.astype(jnp.int32)

    def mol_body(mol, _):
        molf = mol.astype(jnp.float32)

        def vec_body(j, carry):
            ax, ay, az, am = carry
            sl = pl.ds(j * _LANES, _LANES)
            keep = feat_v[4, sl] == molf
            m = jnp.where(keep, m_v[sl], 0.0)
            ax = ax + m * feat_v[0, sl]
            ay = ay + m * feat_v[1, sl]
            az = az + m * feat_v[2, sl]
            am = am + m
            return ax, ay, az, am

        z = jnp.zeros((_LANES,), jnp.float32)
        ax, ay, az, am = lax.fori_loop(0, _CHUNK // _LANES, vec_body,
                                       (z, z, z, z))
        acc_v[0, mol, :] = ax
        acc_v[1, mol, :] = ay
        acc_v[2, mol, :] = az
        acc_v[3, mol, :] = am
        return 0

    lax.fori_loop(bmin, bmax + 1, mol_body, 0)

    pltpu.sync_copy(acc_v, out_hbm.at[wid])


def _mass_side(feat, atno1d, masses_table):
    mesh = plsc.VectorSubcoreMesh(core_axis_name="c", subcore_axis_name="s")
    k = functools.partial(
        pl.kernel,
        out_type=jax.ShapeDtypeStruct((_NW, 4, _N_MOL, _LANES),
                                      jnp.float32),
        mesh=mesh,
        scratch_types=[
            pltpu.VMEM((8, _CHUNK), jnp.float32),
            pltpu.VMEM((_CHUNK,), jnp.int32),
            pltpu.VMEM((_CHUNK,), jnp.float32),
            pltpu.VMEM((128,), jnp.float32),
            pltpu.VMEM((4, _N_MOL, _LANES), jnp.float32),
            pltpu.SemaphoreType.DMA,
        ],
    )(_mass_side_kernel)
    return k(feat, atno1d, masses_table)


def _s_side_kernel(x_ref, feat_ref, W1_ref, b1_ref, W2_ref, b2_ref,
                   out_ref):
    i = pl.program_id(0)

    @pl.when(i == 0)
    def _init():
        out_ref[...] = jnp.zeros_like(out_ref)

    x = x_ref[...]
    ft = feat_ref[...]

    px = ft[0:1, :]
    py = ft[1:2, :]
    pz = ft[2:3, :]
    r2 = ft[3:4, :]
    bf = ft[4:5, :]

    bi = bf.astype(jnp.int32)
    seg = jnp.where(
        lax.broadcasted_iota(jnp.int32, (_N_MOL, _TILE), 0) == bi,
        1.0, 0.0)

    h = jnp.dot(x, W1_ref[...], preferred_element_type=jnp.float32)
    h = h + b1_ref[...]
    h = h * jax.nn.sigmoid(h)
    s = jnp.dot(h, W2_ref[...], preferred_element_type=jnp.float32)
    s = s + b2_ref[...]

    SF = jnp.concatenate([seg * r2, seg * px, seg * py, seg * pz, seg],
                         axis=0)
    out_ref[...] += lax.dot_general(
        SF, s, (((1,), (0,)), ((), ())),
        preferred_element_type=jnp.float32)


def _s_side(x_scalar, feat, W1, b1r, W2, b2r):
    return pl.pallas_call(
        _s_side_kernel,
        grid=(_GRID,),
        in_specs=[
            pl.BlockSpec((_TILE, _NODE_DIM), lambda i: (i, 0)),
            pl.BlockSpec((8, _TILE), lambda i: (0, i)),
            pl.BlockSpec((_NODE_DIM, _HIDDEN_DIM), lambda i: (0, 0)),
            pl.BlockSpec((1, _HIDDEN_DIM), lambda i: (0, 0)),
            pl.BlockSpec((_HIDDEN_DIM, 1), lambda i: (0, 0)),
            pl.BlockSpec((1, 1), lambda i: (0, 0)),
        ],
        out_specs=pl.BlockSpec((80, 1), lambda i: (0, 0)),
        out_shape=jax.ShapeDtypeStruct((80, 1), jnp.float32),
        compiler_params=pltpu.CompilerParams(
            dimension_semantics=("arbitrary",)),
    )(x_scalar, feat, W1, b1r, W2, b2r)


def kernel(x_scalar, x_spherical, pos, batch, at_no, masses_table, W1, b1,
           W2, b2):
    del x_spherical
    posT = pos.T
    r2row = jnp.sum(pos * pos, axis=1)[None, :]
    bfrow = batch.astype(jnp.float32)[None, :]
    feat = jnp.concatenate(
        [posT, r2row, bfrow, jnp.zeros((3, _N_ATOMS), jnp.float32)],
        axis=0)
    atno1d = at_no.astype(jnp.int32)
    b1r = b1.reshape(1, _HIDDEN_DIM)
    b2r = b2.reshape(1, 1)

    sc_part = _mass_side(feat, atno1d, masses_table)
    accs = _s_side(x_scalar, feat, W1, b1r, W2, b2r)

    mass = jnp.sum(sc_part, axis=(0, 3))
    A = accs[0:16, 0]
    Bx = accs[16:32, 0]
    By = accs[32:48, 0]
    Bz = accs[48:64, 0]
    C = accs[64:80, 0]
    S = mass[3]
    den = jnp.where(S > 0.0, S, 1.0)
    cx = mass[0] / den
    cy = mass[1] / den
    cz = mass[2] / den
    res = (A - 2.0 * (Bx * cx + By * cy + Bz * cz)
           + C * (cx * cx + cy * cy + cz * cz))
    return res.reshape(_N_MOL, 1)


# speedup vs baseline: 1.9066x; 1.9066x over previous
import jax
import jax.numpy as jnp
from jax.experimental import pallas as pl
from jax.experimental.pallas import tpu as pltpu

_N_ATOMS = 32768
_N_MOL = 16
_NODE_DIM = 128
_HIDDEN_DIM = 64
_N_ELEM = 119
_TILE = 8192
_GRID = _N_ATOMS // _TILE


def _spatial_kernel(x_ref, posT_ref, batch_ref, atno_ref, mt_ref, W1_ref,
                    b1_ref, W2_ref, b2_ref, out_ref, accs_ref, accm_ref):
    i = pl.program_id(0)

    @pl.when(i == 0)
    def _init():
        accs_ref[...] = jnp.zeros_like(accs_ref)
        accm_ref[...] = jnp.zeros_like(accm_ref)

    x = x_ref[...]
    pT = posT_ref[...]
    b = batch_ref[...]
    a = atno_ref[...]

    el = jax.lax.broadcasted_iota(jnp.int32, (128, _TILE), 0)
    m = jnp.sum(jnp.where(el == a, mt_ref[...], 0.0), axis=0,
                keepdims=True)

    px = pT[0:1, :]
    py = pT[1:2, :]
    pz = pT[2:3, :]
    r2 = px * px + py * py + pz * pz

    seg = jnp.where(
        jax.lax.broadcasted_iota(jnp.int32, (_N_MOL, _TILE), 0) == b,
        1.0, 0.0)

    h = jnp.dot(x, W1_ref[...], preferred_element_type=jnp.float32)
    h = h + b1_ref[...]
    h = h * jax.nn.sigmoid(h)
    s = jnp.dot(h, W2_ref[...], preferred_element_type=jnp.float32)
    s = s + b2_ref[...]

    SF = jnp.concatenate([seg * r2, seg * px, seg * py, seg * pz, seg],
                         axis=0)
    accs_ref[...] += jax.lax.dot_general(
        SF, s, (((1,), (0,)), ((), ())),
        preferred_element_type=jnp.float32)

    MF = jnp.concatenate([seg * px, seg * py, seg * pz, seg],
                         axis=0)
    accm_ref[...] += jax.lax.dot_general(
        MF, m, (((1,), (1,)), ((), ())),
        preferred_element_type=jnp.float32)

    @pl.when(i == _GRID - 1)
    def _finalize():
        A = accs_ref[0:16, :]
        Bx = accs_ref[16:32, :]
        By = accs_ref[32:48, :]
        Bz = accs_ref[48:64, :]
        C = accs_ref[64:80, :]
        Mx = accm_ref[0:16, :]
        My = accm_ref[16:32, :]
        Mz = accm_ref[32:48, :]
        S = accm_ref[48:64, :]
        den = jnp.where(S > 0.0, S, 1.0)
        cx = Mx / den
        cy = My / den
        cz = Mz / den
        res = (A - 2.0 * (Bx * cx + By * cy + Bz * cz)
               + C * (cx * cx + cy * cy + cz * cz))
        out_ref[...] = res


def kernel(x_scalar, x_spherical, pos, batch, at_no, masses_table, W1, b1,
           W2, b2):
    del x_spherical
    posT = pos.T
    batch2 = batch.astype(jnp.int32).reshape(1, _N_ATOMS)
    atno2 = at_no.astype(jnp.int32).reshape(1, _N_ATOMS)
    mt = jnp.zeros((128, 1), jnp.float32).at[:_N_ELEM, 0].set(masses_table)
    b1r = b1.reshape(1, _HIDDEN_DIM)
    b2r = b2.reshape(1, 1)

    out = pl.pallas_call(
        _spatial_kernel,
        grid=(_GRID,),
        in_specs=[
            pl.BlockSpec((_TILE, _NODE_DIM), lambda i: (i, 0)),
            pl.BlockSpec((3, _TILE), lambda i: (0, i)),
            pl.BlockSpec((1, _TILE), lambda i: (0, i)),
            pl.BlockSpec((1, _TILE), lambda i: (0, i)),
            pl.BlockSpec((128, 1), lambda i: (0, 0)),
            pl.BlockSpec((_NODE_DIM, _HIDDEN_DIM), lambda i: (0, 0)),
            pl.BlockSpec((1, _HIDDEN_DIM), lambda i: (0, 0)),
            pl.BlockSpec((_HIDDEN_DIM, 1), lambda i: (0, 0)),
            pl.BlockSpec((1, 1), lambda i: (0, 0)),
        ],
        out_specs=pl.BlockSpec((_N_MOL, 1), lambda i: (0, 0)),
        out_shape=jax.ShapeDtypeStruct((_N_MOL, 1), jnp.float32),
        scratch_shapes=[pltpu.VMEM((80, 1), jnp.float32),
                        pltpu.VMEM((64, 1), jnp.float32)],
        compiler_params=pltpu.CompilerParams(
            dimension_semantics=("arbitrary",)),
    )(x_scalar, posT, batch2, atno2, mt, W1, b1r, W2, b2r)
    return out
